# baseline (device time: 53774 ns/iter reference)
import jax
import jax.numpy as jnp
from jax import lax
from jax.experimental import pallas as pl
from jax.experimental.pallas import tpu as pltpu

N_DEV = 8
B_PER = 2
SQ = 256
D_MODEL = 512
H_PER = 4
DH = 64
HG = H_PER * DH
BLK = 64

_R_SRC, _R_DST = (0, 1, 2), (1, 2, 3)
_L_SRC, _L_DST = (0, 7), (7, 6)
_Z_SRC, _Z_DST = (0, 1), (4, 5)

_ROUND_CHUNKS = ((0,), (1, 7, 4), (2, 6, 5), (3,))


def _body(x_ref, wq_ref, wo_ref, k_ref, v_ref, out_ref,
          wqg_ref, wog_ref, ctx_ref,
          qr_send, qr_recv, ql_send, ql_recv, qz_send, qz_recv,
          or_send, or_recv, ol_send, ol_recv, oz_send, oz_recv):
    my = lax.axis_index("i")
    left = lax.rem(my + N_DEV - 1, N_DEV)
    right = lax.rem(my + 1, N_DEV)
    zpeer = lax.rem(my + 4, N_DEV)

    barrier_sem = pltpu.get_barrier_semaphore()
    for nbr in (left, right, zpeer):
        pl.semaphore_signal(
            barrier_sem, inc=1,
            device_id=(nbr,), device_id_type=pl.DeviceIdType.MESH,
        )
    pl.semaphore_wait(barrier_sem, 3)

    wqg_ref[0] = wq_ref[...]
    wog_ref[0] = wo_ref[...]

    x2b = x_ref[...].reshape(B_PER * SQ, D_MODEL).astype(jnp.bfloat16)

    qblk = lax.broadcasted_iota(jnp.int32, (SQ, SQ), 0) // BLK
    kblk = lax.broadcasted_iota(jnp.int32, (SQ, SQ), 1) // BLK
    mask = kblk <= qblk

    def attn_chunk(r):
        q2 = jnp.dot(x2b, wqg_ref[r],
                     preferred_element_type=jnp.float32)
        for b in range(B_PER):
            heads = []
            for h in range(H_PER):
                q = q2[b * SQ:(b + 1) * SQ, h * DH:(h + 1) * DH]
                k = k_ref[r, b, :, h * DH:(h + 1) * DH]
                v = v_ref[r, b, :, h * DH:(h + 1) * DH]
                s = lax.dot_general(
                    q, k, (((1,), (1,)), ((), ())),
                    preferred_element_type=jnp.float32,
                )
                e = jnp.where(mask, jnp.exp(s), 0.0)
                recip = 1.0 / jnp.sum(e, axis=1, keepdims=True)
                ctx = jnp.dot(e, v,
                              preferred_element_type=jnp.float32) * recip
                heads.append(ctx)
            ctx_b = jnp.concatenate(heads, axis=1).astype(jnp.bfloat16)
            ctx_ref[b * SQ:(b + 1) * SQ, r * HG:(r + 1) * HG] = ctx_b

    def stream_copy(buf, src_slot, dst_slot, send_sem, recv_sem, peer):
        return pltpu.make_async_remote_copy(
            src_ref=buf.at[src_slot], dst_ref=buf.at[dst_slot],
            send_sem=send_sem, recv_sem=recv_sem,
            device_id=(peer,), device_id_type=pl.DeviceIdType.MESH,
        )

    for t in range(4):
        started = []
        if t < 3:
            started.append(stream_copy(wqg_ref, _R_SRC[t], _R_DST[t],
                                       qr_send.at[t], qr_recv.at[t], right))
        if t < 2:
            started.append(stream_copy(wqg_ref, _L_SRC[t], _L_DST[t],
                                       ql_send.at[t], ql_recv.at[t], left))
            started.append(stream_copy(wqg_ref, _Z_SRC[t], _Z_DST[t],
                                       qz_send.at[t], qz_recv.at[t], zpeer))
        u = t - 1
        if 0 <= u < 3:
            started.append(stream_copy(wog_ref, _R_SRC[u], _R_DST[u],
                                       or_send.at[u], or_recv.at[u], right))
        if 0 <= u < 2:
            started.append(stream_copy(wog_ref, _L_SRC[u], _L_DST[u],
                                       ol_send.at[u], ol_recv.at[u], left))
            started.append(stream_copy(wog_ref, _Z_SRC[u], _Z_DST[u],
                                       oz_send.at[u], oz_recv.at[u], zpeer))
        for c in started:
            c.start()
        for r in _ROUND_CHUNKS[t]:
            attn_chunk(r)
        for c in started:
            c.wait()

    wo_all = wog_ref[...].reshape(N_DEV * HG, D_MODEL)
    out2 = jnp.dot(ctx_ref[...], wo_all,
                   preferred_element_type=jnp.float32)
    out_ref[...] = out2.reshape(B_PER, SQ, D_MODEL)


def kernel(x, Wq, K_ext, V_ext, Wo):
    my = lax.axis_index("i")

    wq_in = (Wq * 0.125).astype(jnp.bfloat16)
    wo_in = Wo.astype(jnp.bfloat16)

    kb = lax.dynamic_slice_in_dim(K_ext, B_PER * my, B_PER, axis=0)
    vb = lax.dynamic_slice_in_dim(V_ext, B_PER * my, B_PER, axis=0)

    idx = jnp.mod(my - jnp.arange(N_DEV), N_DEV)
    kr = jnp.moveaxis(
        jnp.take(kb.reshape(B_PER, SQ, N_DEV, HG), idx, axis=2), 2, 0)
    vr = jnp.moveaxis(
        jnp.take(vb.reshape(B_PER, SQ, N_DEV, HG), idx, axis=2), 2, 0)

    dma = pltpu.SemaphoreType.DMA
    return pl.pallas_call(
        _body,
        out_shape=jax.ShapeDtypeStruct((B_PER, SQ, D_MODEL), jnp.float32),
        in_specs=[pl.BlockSpec(memory_space=pltpu.VMEM)] * 5,
        out_specs=pl.BlockSpec(memory_space=pltpu.VMEM),
        scratch_shapes=[
            pltpu.VMEM((N_DEV, D_MODEL, HG), jnp.bfloat16),
            pltpu.VMEM((N_DEV, HG, D_MODEL), jnp.bfloat16),
            pltpu.VMEM((B_PER * SQ, N_DEV * HG), jnp.bfloat16),
            dma((3,)), dma((3,)),
            dma((2,)), dma((2,)),
            dma((2,)), dma((2,)),
            dma((3,)), dma((3,)),
            dma((2,)), dma((2,)),
            dma((2,)), dma((2,)),
        ],
        compiler_params=pltpu.CompilerParams(collective_id=0),
    )(x, wq_in, wo_in, kr, vr)
